# baseline (speedup 1.0000x reference)
import functools

import jax
import jax.numpy as jnp
from jax import lax
from jax.experimental import pallas as pl
from jax.experimental.pallas import tpu as pltpu
from jax.experimental.pallas import tpu_sc as plsc

N = 10000
F = 116
FP = 128
H = 256
G = 64
E = 640000

NPAD = 10240
NC, NS = 2, 16
NW = NC * NS
B = 128
NB = 160
CH = 16
NCH = NB // CH
EPT = NB * B
EPAD = EPT * NW
RPT = NPAD // NS
RB = 512
NRB = NPAD // RB

def _deg_body(dst3_hbm, deg_out, didx_all, deg_priv, semi):
    c = lax.axis_index("c")
    s = lax.axis_index("s")
    wid = c * NS + s
    pltpu.async_copy(dst3_hbm.at[wid], didx_all, semi)
    z16 = jnp.zeros((16,), jnp.float32)

    def zbody(i, carry):
        deg_priv[pl.ds(i * 16, 16)] = z16
        return carry

    lax.fori_loop(0, NPAD // 16, zbody, 0)
    pltpu.make_async_copy(dst3_hbm.at[wid], didx_all, semi).wait()

    ones16 = jnp.ones((16,), jnp.float32)

    def body(j, carry):
        for k in range(B // 16):
            idx16 = didx_all[j, pl.ds(k * 16, 16)]
            plsc.addupdate_scatter(deg_priv, [idx16], ones16)
        return carry

    lax.fori_loop(0, NB, body, 0)
    pltpu.sync_copy(deg_priv, deg_out.at[wid])


def _raw_body(src3_hbm, dst3_hbm, xs_hbm, zf_hbm, raw_out,
              sidx_ch, didx_ch, rows2, raw_sh, sem, semi):
    c = lax.axis_index("c")
    s = lax.axis_index("s")
    wid = c * NS + s
    stripe = pl.ds(s * RPT, RPT)
    pltpu.async_copy(src3_hbm.at[wid, pl.ds(0, CH)], sidx_ch.at[0], semi)
    pltpu.async_copy(dst3_hbm.at[wid, pl.ds(0, CH)], didx_ch.at[0], semi)
    pltpu.sync_copy(zf_hbm.at[stripe], raw_sh.at[stripe])
    plsc.subcore_barrier()
    pltpu.make_async_copy(src3_hbm.at[wid, pl.ds(0, CH)], sidx_ch.at[0], semi).wait()
    pltpu.make_async_copy(dst3_hbm.at[wid, pl.ds(0, CH)], didx_ch.at[0], semi).wait()

    def chunk_body(ci, carry):
        cb = ci % 2
        nci = ci + 1

        @pl.when(nci < NCH)
        def _():
            nb2 = nci % 2
            off = pl.multiple_of(nci * CH, 8)
            pltpu.async_copy(src3_hbm.at[wid, pl.ds(off, CH)], sidx_ch.at[nb2], semi)
            pltpu.async_copy(dst3_hbm.at[wid, pl.ds(off, CH)], didx_ch.at[nb2], semi)

        pltpu.async_copy(xs_hbm.at[sidx_ch.at[cb, 0]], rows2.at[0], sem.at[0])

        def body(bj, carry2):
            p = bj % 2
            nxt = bj + 1

            @pl.when(nxt < CH)
            def _():
                q = nxt % 2
                pltpu.async_copy(xs_hbm.at[sidx_ch.at[cb, nxt]], rows2.at[q], sem.at[q])

            pltpu.make_async_copy(xs_hbm.at[sidx_ch.at[cb, bj]], rows2.at[p], sem.at[p]).wait()
            pltpu.sync_copy(rows2.at[p], raw_sh.at[didx_ch.at[cb, bj]], add=True)
            return carry2

        lax.fori_loop(0, CH, body, 0)

        @pl.when(nci < NCH)
        def _():
            nb2 = nci % 2
            off = pl.multiple_of(nci * CH, 8)
            pltpu.make_async_copy(src3_hbm.at[wid, pl.ds(off, CH)], sidx_ch.at[nb2], semi).wait()
            pltpu.make_async_copy(dst3_hbm.at[wid, pl.ds(off, CH)], didx_ch.at[nb2], semi).wait()

        return carry

    lax.fori_loop(0, NCH, chunk_body, 0)
    plsc.subcore_barrier()
    pltpu.sync_copy(raw_sh.at[stripe], raw_out.at[c, stripe])


@functools.cache
def _sc_kernels():
    mesh = plsc.VectorSubcoreMesh(core_axis_name="c", subcore_axis_name="s",
                                  num_cores=NC, num_subcores=NS)
    deg_kernel = pl.kernel(
        _deg_body,
        out_type=jax.ShapeDtypeStruct((NW, NPAD), jnp.float32),
        mesh=mesh,
        scratch_types=[
            pltpu.VMEM((NB, B), jnp.int32),
            pltpu.VMEM((NPAD,), jnp.float32),
            pltpu.SemaphoreType.DMA,
        ],
        compiler_params=pltpu.CompilerParams(needs_layout_passes=False),
    )
    raw_kernel = pl.kernel(
        _raw_body,
        out_type=jax.ShapeDtypeStruct((NC, NPAD, FP), jnp.float32),
        mesh=mesh,
        scratch_types=[
            pltpu.VMEM((2, CH, B), jnp.int32),
            pltpu.VMEM((2, CH, B), jnp.int32),
            pltpu.VMEM((2, B, FP), jnp.float32),
            pltpu.VMEM_SHARED((NPAD, FP), jnp.float32),
            pltpu.SemaphoreType.DMA((2,)),
            pltpu.SemaphoreType.DMA,
        ],
    )
    return deg_kernel, raw_kernel


def _prep_body(dall_ref, x_ref, xs_ref, dinv_ref):
    dsum = lax.dot_general(dall_ref[...], jnp.ones((NW, 1), jnp.float32),
                           (((0,), (0,)), ((), ())),
                           preferred_element_type=jnp.float32)
    dinv = lax.rsqrt(dsum + 1.0)
    dinv_ref[...] = dinv
    xs_ref[...] = x_ref[...] * dinv


_prep = pl.pallas_call(
    _prep_body,
    grid=(NRB,),
    in_specs=[
        pl.BlockSpec((NW, RB), lambda i: (0, i)),
        pl.BlockSpec((RB, FP), lambda i: (i, 0)),
    ],
    out_specs=[
        pl.BlockSpec((RB, FP), lambda i: (i, 0)),
        pl.BlockSpec((RB, 1), lambda i: (i, 0)),
    ],
    out_shape=[
        jax.ShapeDtypeStruct((NPAD, FP), jnp.float32),
        jax.ShapeDtypeStruct((NPAD, 1), jnp.float32),
    ],
)


def _final_body(r0_ref, r1_ref, xs_ref, dinv_ref, b3_ref, W1_ref, b1_ref,
                Wl_ref, bl_ref, xt_ref, pooled_ref, acc, cnt):
    i = pl.program_id(0)

    @pl.when(i == 0)
    def _():
        acc[...] = jnp.zeros_like(acc)
        cnt[...] = jnp.zeros_like(cnt)

    agg = dinv_ref[...] * (r0_ref[...] + r1_ref[...] + xs_ref[...])
    h = jnp.dot(agg, W1_ref[...], preferred_element_type=jnp.float32) + b1_ref[...]
    h = jnp.maximum(h, 0.0)
    b = b3_ref[0, 0, :]
    oh_t = (lax.broadcasted_iota(jnp.int32, (G, RB), 0) == b[None, :]).astype(jnp.float32)
    acc[...] += jnp.dot(oh_t, h, preferred_element_type=jnp.float32)
    cnt[...] += jnp.dot(oh_t, jnp.ones((RB, H), jnp.float32), preferred_element_type=jnp.float32)

    @pl.when(i == pl.num_programs(0) - 1)
    def _():
        pooled = acc[...] / jnp.maximum(cnt[...], 1.0)
        pooled_ref[...] = pooled
        xt_ref[...] = jnp.dot(pooled, Wl_ref[...], preferred_element_type=jnp.float32) + bl_ref[...]


_final = pl.pallas_call(
    _final_body,
    grid=(NRB,),
    in_specs=[
        pl.BlockSpec((RB, FP), lambda i: (i, 0)),
        pl.BlockSpec((RB, FP), lambda i: (i, 0)),
        pl.BlockSpec((RB, FP), lambda i: (i, 0)),
        pl.BlockSpec((RB, 1), lambda i: (i, 0)),
        pl.BlockSpec((1, 1, RB), lambda i: (i, 0, 0)),
        pl.BlockSpec((FP, H), lambda i: (0, 0)),
        pl.BlockSpec((1, H), lambda i: (0, 0)),
        pl.BlockSpec((H, 2), lambda i: (0, 0)),
        pl.BlockSpec((1, 2), lambda i: (0, 0)),
    ],
    out_specs=[
        pl.BlockSpec((G, 2), lambda i: (0, 0)),
        pl.BlockSpec((G, H), lambda i: (0, 0)),
    ],
    out_shape=[
        jax.ShapeDtypeStruct((G, 2), jnp.float32),
        jax.ShapeDtypeStruct((G, H), jnp.float32),
    ],
    scratch_shapes=[
        pltpu.VMEM((G, H), jnp.float32),
        pltpu.VMEM((G, H), jnp.float32),
    ],
)


def kernel(x, edge_index, batch, W1, b1, W_lin, b_lin):
    f32 = jnp.float32
    x_pad = jnp.zeros((NPAD, FP), f32).at[:N, :F].set(x)
    fill = jnp.full((EPAD - E,), NPAD - 1, jnp.int32)
    src3 = jnp.concatenate([edge_index[0], fill]).reshape(NW, NB, B)
    dst3 = jnp.concatenate([edge_index[1], fill]).reshape(NW, NB, B)
    zf = jnp.zeros((NPAD, FP), f32)

    deg_kernel, raw_kernel = _sc_kernels()
    deg_all = deg_kernel(dst3)
    xs, dinv = _prep(deg_all, x_pad)
    raws = raw_kernel(src3, dst3, xs, zf)

    batch_pad = jnp.concatenate([batch, jnp.full((NPAD - N,), G, jnp.int32)])
    batch3 = batch_pad.reshape(NRB, 1, RB)
    W1p = jnp.zeros((FP, H), f32).at[:F].set(W1)

    x_t, pooled = _final(raws[0], raws[1], xs, dinv, batch3, W1p,
                         b1.reshape(1, H), W_lin, b_lin.reshape(1, 2))
    return (x_t, pooled)


# speedup vs baseline: 1.9383x; 1.9383x over previous
import functools

import jax
import jax.numpy as jnp
from jax import lax
from jax.experimental import pallas as pl
from jax.experimental.pallas import tpu as pltpu
from jax.experimental.pallas import tpu_sc as plsc

N = 10000
F = 116
FP = 128
H = 256
G = 64
E = 640000

NPAD = 10240
NC, NS = 2, 16
NW = NC * NS
B = 128
NB = 157
EPT = NB * B
EPAD = EPT * NW
RPT = NPAD // NS
RB = 512
NRB = NPAD // RB

def _deg_body(dst3_hbm, deg_out, didx_all, deg_priv, semi):
    c = lax.axis_index("c")
    s = lax.axis_index("s")
    wid = c * NS + s
    pltpu.async_copy(dst3_hbm.at[wid], didx_all, semi)
    z16 = jnp.zeros((16,), jnp.float32)

    def zbody(i, carry):
        deg_priv[pl.ds(i * 16, 16)] = z16
        return carry

    lax.fori_loop(0, NPAD // 16, zbody, 0)
    pltpu.make_async_copy(dst3_hbm.at[wid], didx_all, semi).wait()

    ones16 = jnp.ones((16,), jnp.float32)

    def body(j, carry):
        for k in range(B // 16):
            idx16 = didx_all[j, pl.ds(k * 16, 16)]
            plsc.addupdate_scatter(deg_priv, [idx16], ones16)
        return carry

    lax.fori_loop(0, NB, body, 0)
    pltpu.sync_copy(deg_priv, deg_out.at[wid])


def _raw_body(src3_hbm, dst3_hbm, xs_hbm, zf_hbm, raw_out,
              sidx4, didx4, rows2, raw_sh, semg, semi4):
    c = lax.axis_index("c")
    s = lax.axis_index("s")
    wid = c * NS + s
    stripe = pl.ds(s * RPT, RPT)
    pltpu.async_copy(src3_hbm.at[wid, 0], sidx4.at[0], semi4.at[0])
    pltpu.async_copy(dst3_hbm.at[wid, 0], didx4.at[0], semi4.at[0])
    pltpu.async_copy(src3_hbm.at[wid, 1], sidx4.at[1], semi4.at[1])
    pltpu.async_copy(dst3_hbm.at[wid, 1], didx4.at[1], semi4.at[1])
    pltpu.sync_copy(zf_hbm.at[stripe], raw_sh.at[stripe])
    plsc.subcore_barrier()
    pltpu.make_async_copy(src3_hbm.at[wid, 0], sidx4.at[0], semi4.at[0]).wait()
    pltpu.make_async_copy(dst3_hbm.at[wid, 0], didx4.at[0], semi4.at[0]).wait()
    pltpu.async_copy(xs_hbm.at[sidx4.at[0]], rows2.at[0], semg.at[0])

    def body(j, carry):
        p = j % 2
        sj = j % 4

        @pl.when(j + 2 < NB)
        def _():
            s2 = (j + 2) % 4
            pltpu.async_copy(src3_hbm.at[wid, j + 2], sidx4.at[s2], semi4.at[s2])
            pltpu.async_copy(dst3_hbm.at[wid, j + 2], didx4.at[s2], semi4.at[s2])

        pltpu.make_async_copy(xs_hbm.at[sidx4.at[sj]], rows2.at[p], semg.at[p]).wait()

        @pl.when(j + 1 < NB)
        def _():
            q = (j + 1) % 2
            s1 = (j + 1) % 4
            pltpu.make_async_copy(src3_hbm.at[wid, j + 1], sidx4.at[s1], semi4.at[s1]).wait()
            pltpu.make_async_copy(dst3_hbm.at[wid, j + 1], didx4.at[s1], semi4.at[s1]).wait()
            pltpu.async_copy(xs_hbm.at[sidx4.at[s1]], rows2.at[q], semg.at[q])

        pltpu.sync_copy(rows2.at[p], raw_sh.at[didx4.at[sj]], add=True)
        return carry

    lax.fori_loop(0, NB, body, 0)
    plsc.subcore_barrier()
    pltpu.sync_copy(raw_sh.at[stripe], raw_out.at[c, stripe])


@functools.cache
def _sc_kernels():
    mesh = plsc.VectorSubcoreMesh(core_axis_name="c", subcore_axis_name="s",
                                  num_cores=NC, num_subcores=NS)
    deg_kernel = pl.kernel(
        _deg_body,
        out_type=jax.ShapeDtypeStruct((NW, NPAD), jnp.float32),
        mesh=mesh,
        scratch_types=[
            pltpu.VMEM((NB, B), jnp.int32),
            pltpu.VMEM((NPAD,), jnp.float32),
            pltpu.SemaphoreType.DMA,
        ],
        compiler_params=pltpu.CompilerParams(needs_layout_passes=False),
    )
    raw_kernel = pl.kernel(
        _raw_body,
        out_type=jax.ShapeDtypeStruct((NC, NPAD, FP), jnp.float32),
        mesh=mesh,
        scratch_types=[
            pltpu.VMEM((4, B), jnp.int32),
            pltpu.VMEM((4, B), jnp.int32),
            pltpu.VMEM((2, B, FP), jnp.float32),
            pltpu.VMEM_SHARED((NPAD, FP), jnp.float32),
            pltpu.SemaphoreType.DMA((2,)),
            pltpu.SemaphoreType.DMA((4,)),
        ],
    )
    return deg_kernel, raw_kernel


def _prep_body(dall_ref, x_ref, xs_ref, dinv_ref):
    dsum = lax.dot_general(dall_ref[...], jnp.ones((NW, 1), jnp.float32),
                           (((0,), (0,)), ((), ())),
                           preferred_element_type=jnp.float32)
    dinv = lax.rsqrt(dsum + 1.0)
    dinv_ref[...] = dinv
    xs_ref[...] = x_ref[...] * dinv


_prep = pl.pallas_call(
    _prep_body,
    grid=(NRB,),
    in_specs=[
        pl.BlockSpec((NW, RB), lambda i: (0, i)),
        pl.BlockSpec((RB, FP), lambda i: (i, 0)),
    ],
    out_specs=[
        pl.BlockSpec((RB, FP), lambda i: (i, 0)),
        pl.BlockSpec((RB, 1), lambda i: (i, 0)),
    ],
    out_shape=[
        jax.ShapeDtypeStruct((NPAD, FP), jnp.float32),
        jax.ShapeDtypeStruct((NPAD, 1), jnp.float32),
    ],
)


def _final_body(r0_ref, r1_ref, xs_ref, dinv_ref, b3_ref, W1_ref, b1_ref,
                Wl_ref, bl_ref, xt_ref, pooled_ref, acc, cnt):
    i = pl.program_id(0)

    @pl.when(i == 0)
    def _():
        acc[...] = jnp.zeros_like(acc)
        cnt[...] = jnp.zeros_like(cnt)

    agg = dinv_ref[...] * (r0_ref[...] + r1_ref[...] + xs_ref[...])
    h = jnp.dot(agg, W1_ref[...], preferred_element_type=jnp.float32) + b1_ref[...]
    h = jnp.maximum(h, 0.0)
    b = b3_ref[0, 0, :]
    oh_t = (lax.broadcasted_iota(jnp.int32, (G, RB), 0) == b[None, :]).astype(jnp.float32)
    acc[...] += jnp.dot(oh_t, h, preferred_element_type=jnp.float32)
    cnt[...] += jnp.dot(oh_t, jnp.ones((RB, H), jnp.float32), preferred_element_type=jnp.float32)

    @pl.when(i == pl.num_programs(0) - 1)
    def _():
        pooled = acc[...] / jnp.maximum(cnt[...], 1.0)
        pooled_ref[...] = pooled
        xt_ref[...] = jnp.dot(pooled, Wl_ref[...], preferred_element_type=jnp.float32) + bl_ref[...]


_final = pl.pallas_call(
    _final_body,
    grid=(NRB,),
    in_specs=[
        pl.BlockSpec((RB, FP), lambda i: (i, 0)),
        pl.BlockSpec((RB, FP), lambda i: (i, 0)),
        pl.BlockSpec((RB, FP), lambda i: (i, 0)),
        pl.BlockSpec((RB, 1), lambda i: (i, 0)),
        pl.BlockSpec((1, 1, RB), lambda i: (i, 0, 0)),
        pl.BlockSpec((FP, H), lambda i: (0, 0)),
        pl.BlockSpec((1, H), lambda i: (0, 0)),
        pl.BlockSpec((H, 2), lambda i: (0, 0)),
        pl.BlockSpec((1, 2), lambda i: (0, 0)),
    ],
    out_specs=[
        pl.BlockSpec((G, 2), lambda i: (0, 0)),
        pl.BlockSpec((G, H), lambda i: (0, 0)),
    ],
    out_shape=[
        jax.ShapeDtypeStruct((G, 2), jnp.float32),
        jax.ShapeDtypeStruct((G, H), jnp.float32),
    ],
    scratch_shapes=[
        pltpu.VMEM((G, H), jnp.float32),
        pltpu.VMEM((G, H), jnp.float32),
    ],
)


def kernel(x, edge_index, batch, W1, b1, W_lin, b_lin):
    f32 = jnp.float32
    x_pad = jnp.zeros((NPAD, FP), f32).at[:N, :F].set(x)
    fill = jnp.full((EPAD - E,), NPAD - 1, jnp.int32)
    src3 = jnp.concatenate([edge_index[0], fill]).reshape(NW, NB, B)
    dst3 = jnp.concatenate([edge_index[1], fill]).reshape(NW, NB, B)
    zf = jnp.zeros((NPAD, FP), f32)

    deg_kernel, raw_kernel = _sc_kernels()
    deg_all = deg_kernel(dst3)
    xs, dinv = _prep(deg_all, x_pad)
    raws = raw_kernel(src3, dst3, xs, zf)

    batch_pad = jnp.concatenate([batch, jnp.full((NPAD - N,), G, jnp.int32)])
    batch3 = batch_pad.reshape(NRB, 1, RB)
    W1p = jnp.zeros((FP, H), f32).at[:F].set(W1)

    x_t, pooled = _final(raws[0], raws[1], xs, dinv, batch3, W1p,
                         b1.reshape(1, H), W_lin, b_lin.reshape(1, 2))
    return (x_t, pooled)


# speedup vs baseline: 2.0622x; 1.0639x over previous
import functools

import jax
import jax.numpy as jnp
from jax import lax
from jax.experimental import pallas as pl
from jax.experimental.pallas import tpu as pltpu
from jax.experimental.pallas import tpu_sc as plsc

N = 10000
F = 116
FP = 128
H = 256
G = 64
E = 640000

NPAD = 10240
NC, NS = 2, 16
NW = NC * NS
B = 128
NB = 157
EPT = NB * B
EPAD = EPT * NW
RPT = NPAD // NS
RB = 512
NRB = NPAD // RB

def _deg_body(dst3_hbm, deg_out, didx_all, deg_priv, semi):
    c = lax.axis_index("c")
    s = lax.axis_index("s")
    wid = c * NS + s
    pltpu.async_copy(dst3_hbm.at[wid], didx_all, semi)
    z16 = jnp.zeros((16,), jnp.float32)

    def zbody(i, carry):
        deg_priv[pl.ds(i * 16, 16)] = z16
        return carry

    lax.fori_loop(0, NPAD // 16, zbody, 0)
    pltpu.make_async_copy(dst3_hbm.at[wid], didx_all, semi).wait()

    ones16 = jnp.ones((16,), jnp.float32)

    def body(j, carry):
        for k in range(B // 16):
            idx16 = didx_all[j, pl.ds(k * 16, 16)]
            plsc.addupdate_scatter(deg_priv, [idx16], ones16)
        return carry

    lax.fori_loop(0, NB, body, 0)
    pltpu.sync_copy(deg_priv, deg_out.at[wid])


def _raw_body(src3_hbm, dst3_hbm, xs_hbm, zf_hbm, raw_out,
              sidx4, didx4, rows2, raw_sh, semg, semi4):
    c = lax.axis_index("c")
    s = lax.axis_index("s")
    wid = c * NS + s
    stripe = pl.ds(s * RPT, RPT)
    pltpu.async_copy(src3_hbm.at[wid, 0], sidx4.at[0], semi4.at[0])
    pltpu.async_copy(dst3_hbm.at[wid, 0], didx4.at[0], semi4.at[0])
    pltpu.async_copy(src3_hbm.at[wid, 1], sidx4.at[1], semi4.at[1])
    pltpu.async_copy(dst3_hbm.at[wid, 1], didx4.at[1], semi4.at[1])
    pltpu.sync_copy(zf_hbm.at[stripe], raw_sh.at[stripe])
    plsc.subcore_barrier()
    pltpu.make_async_copy(src3_hbm.at[wid, 0], sidx4.at[0], semi4.at[0]).wait()
    pltpu.make_async_copy(dst3_hbm.at[wid, 0], didx4.at[0], semi4.at[0]).wait()
    pltpu.async_copy(xs_hbm.at[c].at[sidx4.at[0]], rows2.at[0], semg.at[0])

    def body(j, carry):
        p = j % 2
        sj = j % 4

        @pl.when(j + 2 < NB)
        def _():
            s2 = (j + 2) % 4
            pltpu.async_copy(src3_hbm.at[wid, j + 2], sidx4.at[s2], semi4.at[s2])
            pltpu.async_copy(dst3_hbm.at[wid, j + 2], didx4.at[s2], semi4.at[s2])

        pltpu.make_async_copy(xs_hbm.at[c].at[sidx4.at[sj]], rows2.at[p], semg.at[p]).wait()

        @pl.when(j + 1 < NB)
        def _():
            q = (j + 1) % 2
            s1 = (j + 1) % 4
            pltpu.make_async_copy(src3_hbm.at[wid, j + 1], sidx4.at[s1], semi4.at[s1]).wait()
            pltpu.make_async_copy(dst3_hbm.at[wid, j + 1], didx4.at[s1], semi4.at[s1]).wait()
            pltpu.async_copy(xs_hbm.at[c].at[sidx4.at[s1]], rows2.at[q], semg.at[q])

        pltpu.sync_copy(rows2.at[p], raw_sh.at[didx4.at[sj]], add=True)
        return carry

    lax.fori_loop(0, NB, body, 0)
    plsc.subcore_barrier()
    pltpu.sync_copy(raw_sh.at[stripe], raw_out.at[c, stripe])


@functools.cache
def _sc_kernels():
    mesh = plsc.VectorSubcoreMesh(core_axis_name="c", subcore_axis_name="s",
                                  num_cores=NC, num_subcores=NS)
    deg_kernel = pl.kernel(
        _deg_body,
        out_type=jax.ShapeDtypeStruct((NW, NPAD), jnp.float32),
        mesh=mesh,
        scratch_types=[
            pltpu.VMEM((NB, B), jnp.int32),
            pltpu.VMEM((NPAD,), jnp.float32),
            pltpu.SemaphoreType.DMA,
        ],
        compiler_params=pltpu.CompilerParams(needs_layout_passes=False),
    )
    raw_kernel = pl.kernel(
        _raw_body,
        out_type=jax.ShapeDtypeStruct((NC, NPAD, FP), jnp.float32),
        mesh=mesh,
        scratch_types=[
            pltpu.VMEM((4, B), jnp.int32),
            pltpu.VMEM((4, B), jnp.int32),
            pltpu.VMEM((2, B, FP), jnp.float32),
            pltpu.VMEM_SHARED((NPAD, FP), jnp.float32),
            pltpu.SemaphoreType.DMA((2,)),
            pltpu.SemaphoreType.DMA((4,)),
        ],
    )
    return deg_kernel, raw_kernel


def _prep_body(dall_ref, x_ref, xs_ref, dinv_ref):
    dsum = lax.dot_general(dall_ref[...], jnp.ones((NW, 1), jnp.float32),
                           (((0,), (0,)), ((), ())),
                           preferred_element_type=jnp.float32)
    dinv = lax.rsqrt(dsum + 1.0)
    dinv_ref[...] = dinv
    xsv = x_ref[...] * dinv
    xs_ref[0] = xsv
    xs_ref[1] = xsv


_prep = pl.pallas_call(
    _prep_body,
    grid=(NRB,),
    in_specs=[
        pl.BlockSpec((NW, RB), lambda i: (0, i)),
        pl.BlockSpec((RB, FP), lambda i: (i, 0)),
    ],
    out_specs=[
        pl.BlockSpec((NC, RB, FP), lambda i: (0, i, 0)),
        pl.BlockSpec((RB, 1), lambda i: (i, 0)),
    ],
    out_shape=[
        jax.ShapeDtypeStruct((NC, NPAD, FP), jnp.float32),
        jax.ShapeDtypeStruct((NPAD, 1), jnp.float32),
    ],
)


def _final_body(r0_ref, r1_ref, xs_ref, dinv_ref, b3_ref, W1_ref, b1_ref,
                Wl_ref, bl_ref, xt_ref, pooled_ref, acc, cnt):
    i = pl.program_id(0)

    @pl.when(i == 0)
    def _():
        acc[...] = jnp.zeros_like(acc)
        cnt[...] = jnp.zeros_like(cnt)

    agg = dinv_ref[...] * (r0_ref[...] + r1_ref[...] + xs_ref[...])
    h = jnp.dot(agg, W1_ref[...], preferred_element_type=jnp.float32) + b1_ref[...]
    h = jnp.maximum(h, 0.0)
    b = b3_ref[0, 0, :]
    oh_t = (lax.broadcasted_iota(jnp.int32, (G, RB), 0) == b[None, :]).astype(jnp.float32)
    acc[...] += jnp.dot(oh_t, h, preferred_element_type=jnp.float32)
    cnt[...] += jnp.dot(oh_t, jnp.ones((RB, H), jnp.float32), preferred_element_type=jnp.float32)

    @pl.when(i == pl.num_programs(0) - 1)
    def _():
        pooled = acc[...] / jnp.maximum(cnt[...], 1.0)
        pooled_ref[...] = pooled
        xt_ref[...] = jnp.dot(pooled, Wl_ref[...], preferred_element_type=jnp.float32) + bl_ref[...]


_final = pl.pallas_call(
    _final_body,
    grid=(NRB,),
    in_specs=[
        pl.BlockSpec((RB, FP), lambda i: (i, 0)),
        pl.BlockSpec((RB, FP), lambda i: (i, 0)),
        pl.BlockSpec((RB, FP), lambda i: (i, 0)),
        pl.BlockSpec((RB, 1), lambda i: (i, 0)),
        pl.BlockSpec((1, 1, RB), lambda i: (i, 0, 0)),
        pl.BlockSpec((FP, H), lambda i: (0, 0)),
        pl.BlockSpec((1, H), lambda i: (0, 0)),
        pl.BlockSpec((H, 2), lambda i: (0, 0)),
        pl.BlockSpec((1, 2), lambda i: (0, 0)),
    ],
    out_specs=[
        pl.BlockSpec((G, 2), lambda i: (0, 0)),
        pl.BlockSpec((G, H), lambda i: (0, 0)),
    ],
    out_shape=[
        jax.ShapeDtypeStruct((G, 2), jnp.float32),
        jax.ShapeDtypeStruct((G, H), jnp.float32),
    ],
    scratch_shapes=[
        pltpu.VMEM((G, H), jnp.float32),
        pltpu.VMEM((G, H), jnp.float32),
    ],
)


def kernel(x, edge_index, batch, W1, b1, W_lin, b_lin):
    f32 = jnp.float32
    x_pad = jnp.zeros((NPAD, FP), f32).at[:N, :F].set(x)
    fill = jnp.full((EPAD - E,), NPAD - 1, jnp.int32)
    src3 = jnp.concatenate([edge_index[0], fill]).reshape(NW, NB, B)
    dst3 = jnp.concatenate([edge_index[1], fill]).reshape(NW, NB, B)
    zf = jnp.zeros((NPAD, FP), f32)

    deg_kernel, raw_kernel = _sc_kernels()
    deg_all = deg_kernel(dst3)
    xs, dinv = _prep(deg_all, x_pad)
    raws = raw_kernel(src3, dst3, xs, zf)

    batch_pad = jnp.concatenate([batch, jnp.full((NPAD - N,), G, jnp.int32)])
    batch3 = batch_pad.reshape(NRB, 1, RB)
    W1p = jnp.zeros((FP, H), f32).at[:F].set(W1)

    x_t, pooled = _final(raws[0], raws[1], xs[0], dinv, batch3, W1p,
                         b1.reshape(1, H), W_lin, b_lin.reshape(1, 2))
    return (x_t, pooled)
